# baseline (speedup 1.0000x reference)
import functools

import jax
import jax.numpy as jnp
from jax import lax
from jax.experimental import pallas as pl
from jax.experimental.pallas import tpu as pltpu
from jax.experimental.pallas import tpu_sc as plsc

N_TOK = 4096
D_DIM = 1024
H_DIM = 4096
N_EXP = 8
BLK = 256
BLK_H = 1024
N_HBLK = H_DIM // BLK_H
P_MAX = N_TOK + N_EXP * BLK
N_BLK = P_MAX // BLK

_NC, _NS = 2, 16
NW = _NC * _NS
ROWS_PER_W = N_TOK // NW
CHUNK = 64
N_CHUNK = ROWS_PER_W // CHUNK


def _cumsum_ax0(z, n):
    k = 1
    while k < n:
        pad = jnp.zeros((k,) + z.shape[1:], z.dtype)
        z = z + jnp.concatenate([pad, z[:-k]], axis=0)
        k *= 2
    return z


def _cumsum_ax1(z, n):
    k = 1
    while k < n:
        pad = jnp.zeros(z.shape[:1] + (k,), z.dtype)
        z = z + jnp.concatenate([pad, z[:, :-k]], axis=1)
        k *= 2
    return z


def _meta_body(x_ref, gw_ref, dest_ref, be_ref, bv_ref):
    xv = x_ref[...]
    logits = jnp.dot(xv, gw_ref[...], preferred_element_type=jnp.float32)
    e_iota = lax.broadcasted_iota(jnp.int32, (N_TOK, N_EXP), 1)
    maxv = jnp.max(logits, axis=-1, keepdims=True)
    cand = jnp.where(logits == maxv, e_iota, N_EXP)
    top1 = jnp.min(cand, axis=-1, keepdims=True)
    oh = (e_iota == top1).astype(jnp.int32)
    csum = _cumsum_ax0(oh, N_TOK)
    counts = csum[-1:, :]
    rank = jnp.sum((csum - oh) * oh, axis=1, keepdims=True)
    padded = ((counts + BLK - 1) // BLK) * BLK
    start = _cumsum_ax1(padded, N_EXP) - padded
    dest = jnp.sum(oh * start, axis=1, keepdims=True) + rank
    dest_ref[...] = dest
    gstart = lax.broadcasted_iota(jnp.int32, (N_BLK, N_EXP), 0) * BLK
    ge = (gstart >= jnp.broadcast_to(start, (N_BLK, N_EXP))).astype(jnp.int32)
    be = jnp.clip(jnp.sum(ge, axis=1, keepdims=True) - 1, 0, N_EXP - 1)
    real_end = start + counts
    be_oh = (lax.broadcasted_iota(jnp.int32, (N_BLK, N_EXP), 1) == be)
    end_for_block = jnp.sum(
        jnp.where(be_oh, jnp.broadcast_to(real_end, (N_BLK, N_EXP)), 0),
        axis=1, keepdims=True)
    bv = (gstart[:, :1] < end_for_block).astype(jnp.int32)
    be_ref[...] = be
    bv_ref[...] = bv


def _routing_meta(xt, gate_w):
    dest2, be2, bv2 = pl.pallas_call(
        _meta_body,
        out_shape=[
            jax.ShapeDtypeStruct((N_TOK, 1), jnp.int32),
            jax.ShapeDtypeStruct((N_BLK, 1), jnp.int32),
            jax.ShapeDtypeStruct((N_BLK, 1), jnp.int32),
        ],
    )(xt, gate_w)
    return dest2.reshape(N_TOK), be2.reshape(N_BLK), bv2.reshape(N_BLK)


@functools.lru_cache(maxsize=None)
def _make_sc_scatter():
    mesh = plsc.VectorSubcoreMesh(core_axis_name="c", subcore_axis_name="s")

    @functools.partial(
        pl.kernel,
        mesh=mesh,
        out_type=jax.ShapeDtypeStruct((P_MAX, D_DIM), jnp.float32),
        scratch_types=[
            pltpu.VMEM((CHUNK,), jnp.int32),
            pltpu.VMEM((CHUNK, D_DIM), jnp.float32),
            pltpu.SemaphoreType.DMA,
        ],
    )
    def scatter_k(x_hbm, dest_hbm, xg_hbm, idx_v, rows_v, sem):
        wid = lax.axis_index("s") * _NC + lax.axis_index("c")
        base = wid * ROWS_PER_W
        for j in range(N_CHUNK):
            off = base + j * CHUNK
            pltpu.sync_copy(dest_hbm.at[pl.ds(off, CHUNK)], idx_v)
            pltpu.sync_copy(x_hbm.at[pl.ds(off, CHUNK)], rows_v)
            pltpu.async_copy(rows_v, xg_hbm.at[idx_v], sem).wait()

    return scatter_k


@functools.lru_cache(maxsize=None)
def _make_sc_gather():
    mesh = plsc.VectorSubcoreMesh(core_axis_name="c", subcore_axis_name="s")

    @functools.partial(
        pl.kernel,
        mesh=mesh,
        out_type=jax.ShapeDtypeStruct((N_TOK, D_DIM), jnp.float32),
        scratch_types=[
            pltpu.VMEM((CHUNK,), jnp.int32),
            pltpu.VMEM((CHUNK, D_DIM), jnp.float32),
            pltpu.SemaphoreType.DMA,
        ],
    )
    def gather_k(yg_hbm, dest_hbm, y_hbm, idx_v, rows_v, sem):
        wid = lax.axis_index("s") * _NC + lax.axis_index("c")
        base = wid * ROWS_PER_W
        for j in range(N_CHUNK):
            off = base + j * CHUNK
            pltpu.sync_copy(dest_hbm.at[pl.ds(off, CHUNK)], idx_v)
            pltpu.async_copy(yg_hbm.at[idx_v], rows_v, sem).wait()
            pltpu.sync_copy(rows_v, y_hbm.at[pl.ds(off, CHUNK)])

    return gather_k


def _gelu(x):
    return x * 0.5 * (1.0 + lax.erf(x * (2.0 ** -0.5)))


def _gemm_body(be_ref, bv_ref, xg_ref, lng_ref, lnb_ref, w1_ref, b1_ref,
               w2_ref, b2_ref, out_ref, w1b_ref, w2b_ref):
    h = pl.program_id(0)
    g = pl.program_id(1)
    e = be_ref[g]
    prev_e = be_ref[jnp.maximum(g - 1, 0)]

    @pl.when(jnp.logical_or(g == 0, e != prev_e))
    def _():
        w1b_ref[...] = w1_ref[0].astype(jnp.bfloat16)
        w2b_ref[...] = w2_ref[0].astype(jnp.bfloat16)

    @pl.when(bv_ref[g] > 0)
    def _():
        xb = xg_ref[...]
        mu = jnp.mean(xb, axis=-1, keepdims=True)
        var = jnp.mean((xb - mu) ** 2, axis=-1, keepdims=True)
        xn = (xb - mu) * lax.rsqrt(var + 1e-5) * lng_ref[0] + lnb_ref[0]
        hb = jnp.dot(xn.astype(jnp.bfloat16), w1b_ref[...],
                     preferred_element_type=jnp.float32) + b1_ref[0]
        hb = _gelu(hb)
        contrib = jnp.dot(hb.astype(jnp.bfloat16), w2b_ref[...],
                          preferred_element_type=jnp.float32)
        row0 = g * BLK

        @pl.when(h == 0)
        def _():
            out_ref[pl.ds(row0, BLK), :] = contrib + b2_ref[0]

        @pl.when(h > 0)
        def _():
            out_ref[pl.ds(row0, BLK), :] = out_ref[pl.ds(row0, BLK), :] + contrib


def _grouped_gemm(block_expert, block_valid, xg, ln_g, ln_b, W1, b1, W2, b2):
    grid_spec = pltpu.PrefetchScalarGridSpec(
        num_scalar_prefetch=2,
        grid=(N_HBLK, N_BLK),
        in_specs=[
            pl.BlockSpec((BLK, D_DIM), lambda h, g, be, bv: (g, 0)),
            pl.BlockSpec((1, 1, D_DIM), lambda h, g, be, bv: (be[g], 0, 0)),
            pl.BlockSpec((1, 1, D_DIM), lambda h, g, be, bv: (be[g], 0, 0)),
            pl.BlockSpec((1, D_DIM, BLK_H), lambda h, g, be, bv: (be[g], 0, h)),
            pl.BlockSpec((1, 1, BLK_H), lambda h, g, be, bv: (be[g], 0, h)),
            pl.BlockSpec((1, BLK_H, D_DIM), lambda h, g, be, bv: (be[g], h, 0)),
            pl.BlockSpec((1, 1, D_DIM), lambda h, g, be, bv: (be[g], 0, 0)),
        ],
        out_specs=pl.BlockSpec((P_MAX, D_DIM), lambda h, g, be, bv: (0, 0)),
        scratch_shapes=[
            pltpu.VMEM((D_DIM, BLK_H), jnp.bfloat16),
            pltpu.VMEM((BLK_H, D_DIM), jnp.bfloat16),
        ],
    )
    return pl.pallas_call(
        _gemm_body,
        grid_spec=grid_spec,
        out_shape=jax.ShapeDtypeStruct((P_MAX, D_DIM), jnp.float32),
    )(block_expert, block_valid, xg,
      ln_g.reshape(N_EXP, 1, D_DIM), ln_b.reshape(N_EXP, 1, D_DIM),
      W1, b1.reshape(N_EXP, 1, H_DIM), W2, b2.reshape(N_EXP, 1, D_DIM))


def kernel(x, gate_w, ln_g, ln_b, W1, b1, W2, b2):
    Bb, Tt, Dd = x.shape
    xt = x.reshape(Bb * Tt, Dd)

    dest, be, bv = _routing_meta(xt, gate_w)

    xg = _make_sc_scatter()(xt, dest)
    yg = _grouped_gemm(be, bv, xg, ln_g, ln_b, W1, b1, W2, b2)
    y = _make_sc_gather()(yg, dest)
    return y.reshape(Bb, Tt, Dd)


# speedup vs baseline: 1.1754x; 1.1754x over previous
import functools

import jax
import jax.numpy as jnp
from jax import lax
from jax.experimental import pallas as pl
from jax.experimental.pallas import tpu as pltpu
from jax.experimental.pallas import tpu_sc as plsc

N_TOK = 4096
D_DIM = 1024
H_DIM = 4096
N_EXP = 8
BLK = 256
BLK_H = 1024
N_HBLK = H_DIM // BLK_H
P_MAX = N_TOK + N_EXP * BLK
N_BLK = P_MAX // BLK

_NC, _NS = 2, 16
NW = _NC * _NS
ROWS_PER_W = N_TOK // NW
CHUNK = 64
N_CHUNK = ROWS_PER_W // CHUNK
SUBCHUNK = 32


def _cumsum_ax0(z, n):
    k = 1
    while k < n:
        pad = jnp.zeros((k,) + z.shape[1:], z.dtype)
        z = z + jnp.concatenate([pad, z[:-k]], axis=0)
        k *= 2
    return z


def _cumsum_ax1(z, n):
    k = 1
    while k < n:
        pad = jnp.zeros(z.shape[:1] + (k,), z.dtype)
        z = z + jnp.concatenate([pad, z[:, :-k]], axis=1)
        k *= 2
    return z


def _meta_body(x_ref, gw_ref, dest_ref, be_ref, bv_ref):
    xv = x_ref[...]
    logits = jnp.dot(xv, gw_ref[...], preferred_element_type=jnp.float32)
    e_iota = lax.broadcasted_iota(jnp.int32, (N_TOK, N_EXP), 1)
    maxv = jnp.max(logits, axis=-1, keepdims=True)
    cand = jnp.where(logits == maxv, e_iota, N_EXP)
    top1 = jnp.min(cand, axis=-1, keepdims=True)
    oh = (e_iota == top1).astype(jnp.int32)
    csum = _cumsum_ax0(oh, N_TOK)
    counts = csum[-1:, :]
    rank = jnp.sum((csum - oh) * oh, axis=1, keepdims=True)
    padded = ((counts + BLK - 1) // BLK) * BLK
    start = _cumsum_ax1(padded, N_EXP) - padded
    dest = jnp.sum(oh * start, axis=1, keepdims=True) + rank
    dest_ref[...] = dest
    gstart = lax.broadcasted_iota(jnp.int32, (N_BLK, N_EXP), 0) * BLK
    ge = (gstart >= jnp.broadcast_to(start, (N_BLK, N_EXP))).astype(jnp.int32)
    be = jnp.clip(jnp.sum(ge, axis=1, keepdims=True) - 1, 0, N_EXP - 1)
    real_end = start + counts
    be_oh = (lax.broadcasted_iota(jnp.int32, (N_BLK, N_EXP), 1) == be)
    end_for_block = jnp.sum(
        jnp.where(be_oh, jnp.broadcast_to(real_end, (N_BLK, N_EXP)), 0),
        axis=1, keepdims=True)
    bv = (gstart[:, :1] < end_for_block).astype(jnp.int32)
    be_ref[...] = be
    bv_ref[...] = bv


def _routing_meta(xt, gate_w):
    dest2, be2, bv2 = pl.pallas_call(
        _meta_body,
        out_shape=[
            jax.ShapeDtypeStruct((N_TOK, 1), jnp.int32),
            jax.ShapeDtypeStruct((N_BLK, 1), jnp.int32),
            jax.ShapeDtypeStruct((N_BLK, 1), jnp.int32),
        ],
    )(xt, gate_w)
    return dest2.reshape(N_TOK), be2.reshape(N_BLK), bv2.reshape(N_BLK)


@functools.lru_cache(maxsize=None)
def _make_sc_scatter():
    mesh = plsc.VectorSubcoreMesh(core_axis_name="c", subcore_axis_name="s")
    n_sub = ROWS_PER_W // SUBCHUNK

    @functools.partial(
        pl.kernel,
        mesh=mesh,
        out_type=jax.ShapeDtypeStruct((P_MAX, D_DIM), jnp.float32),
        scratch_types=[
            pltpu.VMEM((SUBCHUNK,), jnp.int32),
            pltpu.VMEM((SUBCHUNK,), jnp.int32),
            pltpu.VMEM((SUBCHUNK, D_DIM), jnp.float32),
            pltpu.VMEM((SUBCHUNK, D_DIM), jnp.float32),
            pltpu.SemaphoreType.DMA,
            pltpu.SemaphoreType.DMA,
            pltpu.SemaphoreType.DMA,
            pltpu.SemaphoreType.DMA,
        ],
    )
    def scatter_k(x_hbm, dest_hbm, xg_hbm, idx0, idx1, rows0, rows1,
                  rs0, rs1, ss0, ss1):
        wid = lax.axis_index("s") * _NC + lax.axis_index("c")
        base = wid * ROWS_PER_W
        idxs, rows = [idx0, idx1], [rows0, rows1]
        rsems, ssems = [rs0, rs1], [ss0, ss1]

        def read(j):
            off = base + j * SUBCHUNK
            b = j % 2
            pltpu.sync_copy(dest_hbm.at[pl.ds(off, SUBCHUNK)], idxs[b])
            pltpu.make_async_copy(
                x_hbm.at[pl.ds(off, SUBCHUNK)], rows[b], rsems[b]).start()

        read(0)
        for j in range(n_sub):
            b = j % 2
            pltpu.make_async_copy(
                x_hbm.at[pl.ds(base + j * SUBCHUNK, SUBCHUNK)],
                rows[b], rsems[b]).wait()
            scat = pltpu.make_async_copy(rows[b], xg_hbm.at[idxs[b]], ssems[b])
            scat.start()
            if j + 1 < n_sub:
                if j >= 1:
                    pltpu.make_async_copy(
                        rows[1 - b], xg_hbm.at[idxs[1 - b]],
                        ssems[1 - b]).wait()
                read(j + 1)
        for j in (n_sub - 2, n_sub - 1):
            b = j % 2
            pltpu.make_async_copy(rows[b], xg_hbm.at[idxs[b]], ssems[b]).wait()

    return scatter_k


@functools.lru_cache(maxsize=None)
def _make_sc_gather():
    mesh = plsc.VectorSubcoreMesh(core_axis_name="c", subcore_axis_name="s")

    @functools.partial(
        pl.kernel,
        mesh=mesh,
        out_type=jax.ShapeDtypeStruct((N_TOK, D_DIM), jnp.float32),
        scratch_types=[
            pltpu.VMEM((CHUNK,), jnp.int32),
            pltpu.VMEM((CHUNK, D_DIM), jnp.float32),
            pltpu.SemaphoreType.DMA,
        ],
    )
    def gather_k(yg_hbm, dest_hbm, y_hbm, idx_v, rows_v, sem):
        wid = lax.axis_index("s") * _NC + lax.axis_index("c")
        base = wid * ROWS_PER_W
        for j in range(N_CHUNK):
            off = base + j * CHUNK
            pltpu.sync_copy(dest_hbm.at[pl.ds(off, CHUNK)], idx_v)
            pltpu.async_copy(yg_hbm.at[idx_v], rows_v, sem).wait()
            pltpu.sync_copy(rows_v, y_hbm.at[pl.ds(off, CHUNK)])

    return gather_k


def _gelu(x):
    return x * 0.5 * (1.0 + lax.erf(x * (2.0 ** -0.5)))


def _is_first_of_run(g, be_ref):
    return jnp.logical_or(g == 0, be_ref[g] != be_ref[jnp.maximum(g - 1, 0)])


def _gemm_body(be_ref, bv_ref, xg_ref, lng_ref, lnb_ref, w1_ref, b1_ref,
               w2_ref, b2_ref, out_ref, w1b_ref, w2b_ref, xn_ref):
    g = pl.program_id(0)
    h = pl.program_id(1)

    @pl.when(_is_first_of_run(g, be_ref))
    def _():
        w1b_ref[:, pl.ds(h * BLK_H, BLK_H)] = w1_ref[0].astype(jnp.bfloat16)
        w2b_ref[pl.ds(h * BLK_H, BLK_H), :] = w2_ref[0].astype(jnp.bfloat16)

    @pl.when(bv_ref[g] > 0)
    def _():
        @pl.when(h == 0)
        def _():
            xb = xg_ref[...]
            mu = jnp.mean(xb, axis=-1, keepdims=True)
            var = jnp.mean((xb - mu) ** 2, axis=-1, keepdims=True)
            xn = (xb - mu) * lax.rsqrt(var + 1e-5) * lng_ref[0] + lnb_ref[0]
            xn_ref[...] = xn.astype(jnp.bfloat16)

        hb = jnp.dot(xn_ref[...], w1b_ref[:, pl.ds(h * BLK_H, BLK_H)],
                     preferred_element_type=jnp.float32) + b1_ref[0]
        hb = _gelu(hb)
        contrib = jnp.dot(hb.astype(jnp.bfloat16),
                          w2b_ref[pl.ds(h * BLK_H, BLK_H), :],
                          preferred_element_type=jnp.float32)

        @pl.when(h == 0)
        def _():
            out_ref[...] = contrib + b2_ref[0]

        @pl.when(h > 0)
        def _():
            out_ref[...] = out_ref[...] + contrib


def _w_chunk_idx(g, h, be_ref):
    return jnp.where(_is_first_of_run(g, be_ref), h, N_HBLK - 1)


def _grouped_gemm(block_expert, block_valid, xg, ln_g, ln_b, W1, b1, W2, b2):
    grid_spec = pltpu.PrefetchScalarGridSpec(
        num_scalar_prefetch=2,
        grid=(N_BLK, N_HBLK),
        in_specs=[
            pl.BlockSpec((BLK, D_DIM), lambda g, h, be, bv: (g, 0)),
            pl.BlockSpec((1, 1, D_DIM), lambda g, h, be, bv: (be[g], 0, 0)),
            pl.BlockSpec((1, 1, D_DIM), lambda g, h, be, bv: (be[g], 0, 0)),
            pl.BlockSpec((1, D_DIM, BLK_H),
                         lambda g, h, be, bv: (be[g], 0, _w_chunk_idx(g, h, be))),
            pl.BlockSpec((1, 1, BLK_H), lambda g, h, be, bv: (be[g], 0, h)),
            pl.BlockSpec((1, BLK_H, D_DIM),
                         lambda g, h, be, bv: (be[g], _w_chunk_idx(g, h, be), 0)),
            pl.BlockSpec((1, 1, D_DIM), lambda g, h, be, bv: (be[g], 0, 0)),
        ],
        out_specs=pl.BlockSpec((BLK, D_DIM), lambda g, h, be, bv: (g, 0)),
        scratch_shapes=[
            pltpu.VMEM((D_DIM, H_DIM), jnp.bfloat16),
            pltpu.VMEM((H_DIM, D_DIM), jnp.bfloat16),
            pltpu.VMEM((BLK, D_DIM), jnp.bfloat16),
        ],
    )
    return pl.pallas_call(
        _gemm_body,
        grid_spec=grid_spec,
        out_shape=jax.ShapeDtypeStruct((P_MAX, D_DIM), jnp.float32),
    )(block_expert, block_valid, xg,
      ln_g.reshape(N_EXP, 1, D_DIM), ln_b.reshape(N_EXP, 1, D_DIM),
      W1, b1.reshape(N_EXP, 1, H_DIM), W2, b2.reshape(N_EXP, 1, D_DIM))


def kernel(x, gate_w, ln_g, ln_b, W1, b1, W2, b2):
    Bb, Tt, Dd = x.shape
    xt = x.reshape(Bb * Tt, Dd)

    dest, be, bv = _routing_meta(xt, gate_w)

    xg = _make_sc_scatter()(xt, dest)
    yg = _grouped_gemm(be, bv, xg, ln_g, ln_b, W1, b1, W2, b2)
    y = _make_sc_gather()(yg, dest)
    return y.reshape(Bb, Tt, Dd)


# speedup vs baseline: 1.1858x; 1.0089x over previous
import functools

import jax
import jax.numpy as jnp
from jax import lax
from jax.experimental import pallas as pl
from jax.experimental.pallas import tpu as pltpu
from jax.experimental.pallas import tpu_sc as plsc

N_TOK = 4096
D_DIM = 1024
H_DIM = 4096
N_EXP = 8
BLK = 256
BLK_H = 1024
N_HBLK = H_DIM // BLK_H
P_MAX = N_TOK + N_EXP * BLK
N_BLK = P_MAX // BLK

_NC, _NS = 2, 16
NW = _NC * _NS
ROWS_PER_W = N_TOK // NW
CHUNK = 64
N_CHUNK = ROWS_PER_W // CHUNK


def _cumsum_ax0(z, n):
    k = 1
    while k < n:
        pad = jnp.zeros((k,) + z.shape[1:], z.dtype)
        z = z + jnp.concatenate([pad, z[:-k]], axis=0)
        k *= 2
    return z


def _cumsum_ax1(z, n):
    k = 1
    while k < n:
        pad = jnp.zeros(z.shape[:1] + (k,), z.dtype)
        z = z + jnp.concatenate([pad, z[:, :-k]], axis=1)
        k *= 2
    return z


def _meta_body(x_ref, gw_ref, dest_ref, be_ref, bv_ref):
    xv = x_ref[...]
    logits = jnp.dot(xv, gw_ref[...], preferred_element_type=jnp.float32)
    e_iota = lax.broadcasted_iota(jnp.int32, (N_TOK, N_EXP), 1)
    maxv = jnp.max(logits, axis=-1, keepdims=True)
    cand = jnp.where(logits == maxv, e_iota, N_EXP)
    top1 = jnp.min(cand, axis=-1, keepdims=True)
    oh = (e_iota == top1).astype(jnp.int32)
    csum = _cumsum_ax0(oh, N_TOK)
    counts = csum[-1:, :]
    rank = jnp.sum((csum - oh) * oh, axis=1, keepdims=True)
    padded = ((counts + BLK - 1) // BLK) * BLK
    start = _cumsum_ax1(padded, N_EXP) - padded
    dest = jnp.sum(oh * start, axis=1, keepdims=True) + rank
    dest_ref[...] = dest
    gstart = lax.broadcasted_iota(jnp.int32, (N_BLK, N_EXP), 0) * BLK
    ge = (gstart >= jnp.broadcast_to(start, (N_BLK, N_EXP))).astype(jnp.int32)
    be = jnp.clip(jnp.sum(ge, axis=1, keepdims=True) - 1, 0, N_EXP - 1)
    real_end = start + counts
    be_oh = (lax.broadcasted_iota(jnp.int32, (N_BLK, N_EXP), 1) == be)
    end_for_block = jnp.sum(
        jnp.where(be_oh, jnp.broadcast_to(real_end, (N_BLK, N_EXP)), 0),
        axis=1, keepdims=True)
    bv = (gstart[:, :1] < end_for_block).astype(jnp.int32)
    be_ref[...] = be
    bv_ref[...] = bv


def _routing_meta(xt, gate_w):
    dest2, be2, bv2 = pl.pallas_call(
        _meta_body,
        out_shape=[
            jax.ShapeDtypeStruct((N_TOK, 1), jnp.int32),
            jax.ShapeDtypeStruct((N_BLK, 1), jnp.int32),
            jax.ShapeDtypeStruct((N_BLK, 1), jnp.int32),
        ],
    )(xt, gate_w)
    return dest2.reshape(N_TOK), be2.reshape(N_BLK), bv2.reshape(N_BLK)


@functools.lru_cache(maxsize=None)
def _make_sc_scatter():
    mesh = plsc.VectorSubcoreMesh(core_axis_name="c", subcore_axis_name="s")

    @functools.partial(
        pl.kernel,
        mesh=mesh,
        out_type=jax.ShapeDtypeStruct((P_MAX, D_DIM), jnp.float32),
        scratch_types=[
            pltpu.VMEM((CHUNK,), jnp.int32),
            pltpu.VMEM((CHUNK, D_DIM), jnp.float32),
            pltpu.SemaphoreType.DMA,
        ],
    )
    def scatter_k(x_hbm, dest_hbm, xg_hbm, idx_v, rows_v, sem):
        wid = lax.axis_index("s") * _NC + lax.axis_index("c")
        base = wid * ROWS_PER_W
        for j in range(N_CHUNK):
            off = base + j * CHUNK
            pltpu.sync_copy(dest_hbm.at[pl.ds(off, CHUNK)], idx_v)
            pltpu.sync_copy(x_hbm.at[pl.ds(off, CHUNK)], rows_v)
            pltpu.async_copy(rows_v, xg_hbm.at[idx_v], sem).wait()

    return scatter_k


@functools.lru_cache(maxsize=None)
def _make_sc_gather():
    mesh = plsc.VectorSubcoreMesh(core_axis_name="c", subcore_axis_name="s")

    @functools.partial(
        pl.kernel,
        mesh=mesh,
        out_type=jax.ShapeDtypeStruct((N_TOK, D_DIM), jnp.float32),
        scratch_types=[
            pltpu.VMEM((CHUNK,), jnp.int32),
            pltpu.VMEM((CHUNK, D_DIM), jnp.float32),
            pltpu.SemaphoreType.DMA,
        ],
    )
    def gather_k(yg_hbm, dest_hbm, y_hbm, idx_v, rows_v, sem):
        wid = lax.axis_index("s") * _NC + lax.axis_index("c")
        base = wid * ROWS_PER_W
        for j in range(N_CHUNK):
            off = base + j * CHUNK
            pltpu.sync_copy(dest_hbm.at[pl.ds(off, CHUNK)], idx_v)
            pltpu.async_copy(yg_hbm.at[idx_v], rows_v, sem).wait()
            pltpu.sync_copy(rows_v, y_hbm.at[pl.ds(off, CHUNK)])

    return gather_k


def _gelu(x):
    return x * 0.5 * (1.0 + lax.erf(x * (2.0 ** -0.5)))


def _is_first_of_run(g, be_ref):
    return jnp.logical_or(g == 0, be_ref[g] != be_ref[jnp.maximum(g - 1, 0)])


def _gemm_body(be_ref, bv_ref, xg_ref, lng_ref, lnb_ref, w1_ref, b1_ref,
               w2_ref, b2_ref, out_ref, w1b_ref, w2b_ref, xn_ref):
    g = pl.program_id(0)
    h = pl.program_id(1)

    @pl.when(_is_first_of_run(g, be_ref))
    def _():
        w1b_ref[:, pl.ds(h * BLK_H, BLK_H)] = w1_ref[0].astype(jnp.bfloat16)
        w2b_ref[pl.ds(h * BLK_H, BLK_H), :] = w2_ref[0].astype(jnp.bfloat16)

    @pl.when(bv_ref[g] > 0)
    def _():
        @pl.when(h == 0)
        def _():
            xb = xg_ref[...]
            mu = jnp.mean(xb, axis=-1, keepdims=True)
            var = jnp.mean((xb - mu) ** 2, axis=-1, keepdims=True)
            xn = (xb - mu) * lax.rsqrt(var + 1e-5) * lng_ref[0] + lnb_ref[0]
            xn_ref[...] = xn.astype(jnp.bfloat16)

        hb = jnp.dot(xn_ref[...], w1b_ref[:, pl.ds(h * BLK_H, BLK_H)],
                     preferred_element_type=jnp.float32) + b1_ref[0]
        hb = _gelu(hb)
        contrib = jnp.dot(hb.astype(jnp.bfloat16),
                          w2b_ref[pl.ds(h * BLK_H, BLK_H), :],
                          preferred_element_type=jnp.float32)

        @pl.when(h == 0)
        def _():
            out_ref[...] = contrib + b2_ref[0]

        @pl.when(h > 0)
        def _():
            out_ref[...] = out_ref[...] + contrib


def _w_chunk_idx(g, h, be_ref):
    return jnp.where(_is_first_of_run(g, be_ref), h, N_HBLK - 1)


def _grouped_gemm(block_expert, block_valid, xg, ln_g, ln_b, W1, b1, W2, b2):
    grid_spec = pltpu.PrefetchScalarGridSpec(
        num_scalar_prefetch=2,
        grid=(N_BLK, N_HBLK),
        in_specs=[
            pl.BlockSpec((BLK, D_DIM), lambda g, h, be, bv: (g, 0)),
            pl.BlockSpec((1, 1, D_DIM), lambda g, h, be, bv: (be[g], 0, 0)),
            pl.BlockSpec((1, 1, D_DIM), lambda g, h, be, bv: (be[g], 0, 0)),
            pl.BlockSpec((1, D_DIM, BLK_H),
                         lambda g, h, be, bv: (be[g], 0, _w_chunk_idx(g, h, be))),
            pl.BlockSpec((1, 1, BLK_H), lambda g, h, be, bv: (be[g], 0, h)),
            pl.BlockSpec((1, BLK_H, D_DIM),
                         lambda g, h, be, bv: (be[g], _w_chunk_idx(g, h, be), 0)),
            pl.BlockSpec((1, 1, D_DIM), lambda g, h, be, bv: (be[g], 0, 0)),
        ],
        out_specs=pl.BlockSpec((BLK, D_DIM), lambda g, h, be, bv: (g, 0)),
        scratch_shapes=[
            pltpu.VMEM((D_DIM, H_DIM), jnp.bfloat16),
            pltpu.VMEM((H_DIM, D_DIM), jnp.bfloat16),
            pltpu.VMEM((BLK, D_DIM), jnp.bfloat16),
        ],
    )
    return pl.pallas_call(
        _gemm_body,
        grid_spec=grid_spec,
        out_shape=jax.ShapeDtypeStruct((P_MAX, D_DIM), jnp.float32),
    )(block_expert, block_valid, xg,
      ln_g.reshape(N_EXP, 1, D_DIM), ln_b.reshape(N_EXP, 1, D_DIM),
      W1, b1.reshape(N_EXP, 1, H_DIM), W2, b2.reshape(N_EXP, 1, D_DIM))


def kernel(x, gate_w, ln_g, ln_b, W1, b1, W2, b2):
    Bb, Tt, Dd = x.shape
    xt = x.reshape(Bb * Tt, Dd)

    dest, be, bv = _routing_meta(xt, gate_w)

    xg = _make_sc_scatter()(xt, dest)
    yg = _grouped_gemm(be, bv, xg, ln_g, ln_b, W1, b1, W2, b2)
    y = _make_sc_gather()(yg, dest)
    return y.reshape(Bb, Tt, Dd)
